# baseline (device time: 307646 ns/iter reference)
import functools

import jax
import jax.numpy as jnp
from jax import lax
from jax.experimental import pallas as pl
from jax.experimental.pallas import tpu as pltpu


def kernel(Q, K, V):
    b, sq, h, d = Q.shape
    skv = K.shape[1]
    scale = d ** -0.5
    r = skv * h

    Qr = Q.reshape(b, h, d)
    Kr = K.reshape(b, r, d)
    Vr = V.reshape(b, r, d)

    def body(q_ref, k_ref, v_ref, o_ref, ul_send, ul_recv, send_sem, recv_sem):
        i = pl.program_id(0)
        q2 = q_ref[0]
        k2 = k_ref[0]
        v2 = v_ref[0]

        s_all = lax.dot_general(
            k2, q2,
            dimension_numbers=(((1,), (1,)), ((), ())),
            preferred_element_type=jnp.float32,
        ) * scale
        ri = lax.broadcasted_iota(jnp.int32, (r, h), 0)
        hi = lax.broadcasted_iota(jnp.int32, (r, h), 1)
        pm = jnp.where(ri % h == hi, jnp.exp(s_all), 0.0)

        p64 = lax.dot_general(
            pm, jnp.ones((h, d), jnp.float32),
            dimension_numbers=(((1,), (0,)), ((), ())),
            preferred_element_type=jnp.float32,
        )
        contrib = p64 * v2
        mask16 = (lax.broadcasted_iota(jnp.int32, (h, r), 1) % h
                  == lax.broadcasted_iota(jnp.int32, (h, r), 0)
                  ).astype(jnp.float32)
        ul_send[i] = lax.dot_general(
            mask16, jnp.concatenate([contrib, p64], axis=1),
            dimension_numbers=(((1,), (0,)), ((), ())),
            preferred_element_type=jnp.float32,
        )

        @pl.when(i == b - 1)
        def _():
            my_x = lax.axis_index("x")
            my_y = lax.axis_index("y")
            my_z = lax.axis_index("z")
            partner = (1 - my_x, my_y, my_z)

            barrier = pltpu.get_barrier_semaphore()
            pl.semaphore_signal(
                barrier, inc=1,
                device_id=partner, device_id_type=pl.DeviceIdType.MESH,
            )
            pl.semaphore_wait(barrier, 1)

            rdma = pltpu.make_async_remote_copy(
                src_ref=ul_send, dst_ref=ul_recv,
                send_sem=send_sem, recv_sem=recv_sem,
                device_id=partner, device_id_type=pl.DeviceIdType.MESH,
            )
            rdma.start()
            rdma.wait()

            tot = ul_send[...] + ul_recv[...]
            o_ref[...] = tot[:, :, :d] / tot[:, :, d:]

    O = pl.pallas_call(
        body,
        grid=(b,),
        in_specs=[
            pl.BlockSpec((1, h, d), lambda i: (i, 0, 0)),
            pl.BlockSpec((1, r, d), lambda i: (i, 0, 0)),
            pl.BlockSpec((1, r, d), lambda i: (i, 0, 0)),
        ],
        out_specs=pl.BlockSpec((b, h, d), lambda i: (0, 0, 0)),
        out_shape=jax.ShapeDtypeStruct((b, h, d), jnp.float32),
        scratch_shapes=[
            pltpu.VMEM((b, h, 2 * d), jnp.float32),
            pltpu.VMEM((b, h, 2 * d), jnp.float32),
            pltpu.SemaphoreType.DMA,
            pltpu.SemaphoreType.DMA,
        ],
        compiler_params=pltpu.CompilerParams(
            collective_id=0,
            dimension_semantics=("arbitrary",),
            vmem_limit_bytes=100 * 1024 * 1024,
        ),
    )(Qr, Kr, Vr)
    return O.reshape(b, sq, h, d)


# device time: 207830 ns/iter; 1.4803x vs baseline; 1.4803x over previous
import jax
import jax.numpy as jnp
from jax import lax
from jax.experimental import pallas as pl
from jax.experimental.pallas import tpu as pltpu


def kernel(Q, K, V):
    b, sq, h, d = Q.shape
    skv = K.shape[1]
    scale = d ** -0.5

    hd = h * d
    hpb = 8
    width = hpb * d
    bf16 = jnp.bfloat16
    Qr = Q.astype(bf16).reshape(b, sq, hd)
    Kr = K.astype(bf16).reshape(b, skv, hd)
    Vr = V.astype(bf16).reshape(b, skv, hd)

    def partial_body(q_ref, k_ref, v_ref, u_ref, l_ref):
        q2 = q_ref[0]
        k2 = k_ref[0]
        v2 = v_ref[0]
        ci = lax.broadcasted_iota(jnp.int32, (hpb, width), 1)
        hi = lax.broadcasted_iota(jnp.int32, (hpb, width), 0)
        mask = (ci // d == hi).astype(bf16)
        qt = q2 * mask
        s = lax.dot_general(
            k2, qt,
            dimension_numbers=(((1,), (1,)), ((), ())),
            preferred_element_type=jnp.float32,
        ) * scale
        p = jnp.exp(s).astype(bf16)
        pexp = lax.dot_general(
            p, mask,
            dimension_numbers=(((1,), (0,)), ((), ())),
            preferred_element_type=jnp.float32,
        )
        u_ref[0] = jnp.sum(pexp * v2.astype(jnp.float32), axis=0,
                           keepdims=True)
        l_ref[0] = jnp.sum(pexp, axis=0, keepdims=True)

    U, L = pl.pallas_call(
        partial_body,
        grid=(b, h // hpb),
        in_specs=[
            pl.BlockSpec((1, sq, width), lambda i, j: (i, 0, j)),
            pl.BlockSpec((1, skv, width), lambda i, j: (i, 0, j)),
            pl.BlockSpec((1, skv, width), lambda i, j: (i, 0, j)),
        ],
        out_specs=[
            pl.BlockSpec((1, sq, width), lambda i, j: (i, 0, j)),
            pl.BlockSpec((1, sq, width), lambda i, j: (i, 0, j)),
        ],
        out_shape=[
            jax.ShapeDtypeStruct((b, sq, hd), jnp.float32),
            jax.ShapeDtypeStruct((b, sq, hd), jnp.float32),
        ],
    )(Qr, Kr, Vr)

    def merge_body(u_ref, l_ref, o_ref, u_peer, l_peer, send_sems, recv_sems):
        my_x = lax.axis_index("x")
        my_y = lax.axis_index("y")
        my_z = lax.axis_index("z")
        partner = (1 - my_x, my_y, my_z)

        barrier = pltpu.get_barrier_semaphore()
        pl.semaphore_signal(
            barrier, inc=1,
            device_id=partner, device_id_type=pl.DeviceIdType.MESH,
        )
        pl.semaphore_wait(barrier, 1)

        ru = pltpu.make_async_remote_copy(
            src_ref=u_ref, dst_ref=u_peer,
            send_sem=send_sems.at[0], recv_sem=recv_sems.at[0],
            device_id=partner, device_id_type=pl.DeviceIdType.MESH,
        )
        rl = pltpu.make_async_remote_copy(
            src_ref=l_ref, dst_ref=l_peer,
            send_sem=send_sems.at[1], recv_sem=recv_sems.at[1],
            device_id=partner, device_id_type=pl.DeviceIdType.MESH,
        )
        ru.start()
        rl.start()
        ru.wait()
        rl.wait()

        u_tot = u_ref[...] + u_peer[...]
        l_tot = l_ref[...] + l_peer[...]
        o_ref[...] = u_tot / l_tot

    O = pl.pallas_call(
        merge_body,
        in_specs=[
            pl.BlockSpec(memory_space=pltpu.VMEM),
            pl.BlockSpec(memory_space=pltpu.VMEM),
        ],
        out_specs=pl.BlockSpec(memory_space=pltpu.VMEM),
        out_shape=jax.ShapeDtypeStruct((b, sq, hd), jnp.float32),
        scratch_shapes=[
            pltpu.VMEM((b, sq, hd), jnp.float32),
            pltpu.VMEM((b, sq, hd), jnp.float32),
            pltpu.SemaphoreType.DMA((2,)),
            pltpu.SemaphoreType.DMA((2,)),
        ],
        compiler_params=pltpu.CompilerParams(collective_id=0),
    )(U, L)
    return O.reshape(b, sq, h, d)
